# baseline (speedup 1.0000x reference)
import jax
import jax.numpy as jnp
from jax import lax
from jax.experimental import pallas as pl
from jax.experimental.pallas import tpu as pltpu
from jax.experimental.pallas import tpu_sc as plsc

VOCAB = 1000
EMBED = 32
BATCH = 4096
SEQ = 200

NC, NS = 2, 16
NW = NC * NS
N = BATCH * SEQ
PER_W = N // NW
CH = 1024
NSTEPS = PER_W // CH


def _body(idx_hbm, tab_hbm, out_hbm, idx_v, rows_v, tab_sh, sem_idx, sem_gat, sem_out):
    wid = lax.axis_index("s") * NC + lax.axis_index("c")
    base = wid * PER_W

    def idx_copy(g, buf):
        return pltpu.make_async_copy(
            idx_hbm.at[pl.ds(base + g * CH, CH)],
            idx_v.at[pl.ds(buf * CH, CH)],
            sem_idx,
        )

    def gather(g, buf):
        return pltpu.make_async_copy(
            tab_sh.at[idx_v.at[pl.ds(buf * CH, CH)]],
            rows_v.at[pl.ds(buf * CH, CH)],
            sem_gat,
        )

    def out_copy(g, buf):
        return pltpu.make_async_copy(
            rows_v.at[pl.ds(buf * CH, CH)],
            out_hbm.at[pl.ds(base + g * CH, CH)],
            sem_out,
        )

    @pl.when(lax.axis_index("s") == 0)
    def _():
        pltpu.sync_copy(tab_hbm, tab_sh)

    idx_copy(0, 0).start()
    plsc.subcore_barrier()

    def step(g, carry):
        buf = lax.rem(g, 2)
        idx_copy(g, buf).wait()

        @pl.when(g + 1 < NSTEPS)
        def _():
            idx_copy(g + 1, 1 - buf).start()

        gather(g, buf).start()
        gather(g, buf).wait()

        @pl.when(g > 0)
        def _():
            out_copy(g - 1, 1 - buf).wait()

        out_copy(g, buf).start()
        return carry

    lax.fori_loop(0, NSTEPS, step, 0)
    out_copy(NSTEPS - 1, (NSTEPS - 1) % 2).wait()


@jax.jit
def _embed(idx_flat, table):
    mesh = plsc.VectorSubcoreMesh(core_axis_name="c", subcore_axis_name="s")
    run = pl.kernel(
        _body,
        out_type=jax.ShapeDtypeStruct((N, EMBED), jnp.bfloat16),
        mesh=mesh,
        scratch_types=[
            pltpu.VMEM((2 * CH,), jnp.int32),
            pltpu.VMEM((2 * CH, EMBED), jnp.bfloat16),
            pltpu.VMEM_SHARED((VOCAB, EMBED), jnp.bfloat16),
            pltpu.SemaphoreType.DMA,
            pltpu.SemaphoreType.DMA,
            pltpu.SemaphoreType.DMA,
        ],
        compiler_params=pltpu.CompilerParams(use_tc_tiling_on_sc=False),
    )
    return run(idx_flat, table)


def kernel(visit_segments, table):
    idx_flat = visit_segments.reshape(N).astype(jnp.int32)
    out = _embed(idx_flat, table.astype(jnp.bfloat16))
    return out.astype(jnp.float32).reshape(BATCH, SEQ, EMBED)


# speedup vs baseline: 1.5867x; 1.5867x over previous
import jax
import jax.numpy as jnp
from jax import lax
from jax.experimental import pallas as pl
from jax.experimental.pallas import tpu as pltpu
from jax.experimental.pallas import tpu_sc as plsc

VOCAB = 1000
EMBED = 32
BATCH = 4096
SEQ = 200

NC, NS = 2, 16
NW = NC * NS
N = BATCH * SEQ
PER_W = N // NW
CH = 1024
NSTEPS = PER_W // CH


def _body(idx_hbm, tab_hbm, out_hbm, idx_v, rows_v, tab_sh, sem_idx, sem_gat, sem_out):
    wid = lax.axis_index("s") * NC + lax.axis_index("c")
    base = wid * PER_W

    def idx_copy(g, buf):
        return pltpu.make_async_copy(
            idx_hbm.at[pl.ds(base + g * CH, CH)],
            idx_v.at[pl.ds(buf * CH, CH)],
            sem_idx,
        )

    def gather(g, buf):
        return pltpu.make_async_copy(
            tab_sh.at[idx_v.at[pl.ds(buf * CH, CH)]],
            rows_v.at[pl.ds(buf * CH, CH)],
            sem_gat,
        )

    def out_copy(g, buf):
        return pltpu.make_async_copy(
            rows_v.at[pl.ds(buf * CH, CH)],
            out_hbm.at[pl.ds(base + g * CH, CH)],
            sem_out,
        )

    @pl.when(lax.axis_index("s") == 0)
    def _():
        pltpu.sync_copy(tab_hbm, tab_sh)

    idx_copy(0, 0).start()
    plsc.subcore_barrier()

    def step(g, carry):
        buf = lax.rem(g, 2)
        idx_copy(g, buf).wait()

        @pl.when(g + 1 < NSTEPS)
        def _():
            idx_copy(g + 1, 1 - buf).start()

        gather(g, buf).start()
        gather(g, buf).wait()

        @pl.when(g > 0)
        def _():
            out_copy(g - 1, 1 - buf).wait()

        out_copy(g, buf).start()
        return carry

    lax.fori_loop(0, NSTEPS, step, 0)
    out_copy(NSTEPS - 1, (NSTEPS - 1) % 2).wait()


@jax.jit
def _embed(idx_flat, table):
    mesh = plsc.VectorSubcoreMesh(core_axis_name="c", subcore_axis_name="s")
    run = pl.kernel(
        _body,
        out_type=jax.ShapeDtypeStruct((N, EMBED), jnp.float32),
        mesh=mesh,
        scratch_types=[
            pltpu.VMEM((2 * CH,), jnp.int32),
            pltpu.VMEM((2 * CH, EMBED), jnp.float32),
            pltpu.VMEM_SHARED((VOCAB, EMBED), jnp.float32),
            pltpu.SemaphoreType.DMA,
            pltpu.SemaphoreType.DMA,
            pltpu.SemaphoreType.DMA,
        ],
        compiler_params=pltpu.CompilerParams(use_tc_tiling_on_sc=False),
    )
    return run(idx_flat, table)


def kernel(visit_segments, table):
    idx_flat = visit_segments.reshape(N).astype(jnp.int32)
    out = _embed(idx_flat, table)
    return out.reshape(BATCH, SEQ, EMBED)
